# Initial kernel scaffold; baseline (speedup 1.0000x reference)
import jax
import jax.numpy as jnp
from jax.experimental import pallas as pl


def kernel(input_ids, attention_mask, token_type_ids, emb_table, type_table, W, b):
    raise NotImplementedError("write your pallas kernel here")


# speedup vs baseline: 5.3163x; 5.3163x over previous
import functools

import jax
import jax.numpy as jnp
from jax import lax
from jax.experimental import pallas as pl
from jax.experimental.pallas import tpu as pltpu
from jax.experimental.pallas import tpu_sc as plsc

V, H, L = 1000000, 32, 2
B, S = 4096, 200

NC, NS = 2, 16
NW = NC * NS
SPW = B // NW
G = S // 2
HALF = H // 2


def _sc_emb_sum(ids2d, table):
    mesh = plsc.VectorSubcoreMesh(core_axis_name="c", subcore_axis_name="s")

    @functools.partial(
        pl.kernel,
        mesh=mesh,
        out_type=jax.ShapeDtypeStruct((B, H), jnp.float32),
        compiler_params=pltpu.CompilerParams(use_tc_tiling_on_sc=False),
        scratch_types=[
            pltpu.VMEM((2 * SPW, G), jnp.int32),
            pltpu.VMEM((G, H), jnp.float32),
            pltpu.VMEM((G, H), jnp.float32),
            pltpu.VMEM((G, H), jnp.float32),
            pltpu.VMEM((G, H), jnp.float32),
            pltpu.VMEM((SPW, H), jnp.float32),
            pltpu.SemaphoreType.DMA,
            pltpu.SemaphoreType.DMA,
        ],
    )
    def body(ids_hbm, table_hbm, out_hbm, ids_v, r0a, r0b, r1a, r1b, sums_v,
             sem0, sem1):
        wid = lax.axis_index("s") * NC + lax.axis_index("c")
        base = wid * (2 * SPW)
        pltpu.sync_copy(ids_hbm.at[pl.ds(base, 2 * SPW)], ids_v)

        def fire(s, ra, rb, sem):
            pltpu.async_copy(table_hbm.at[ids_v.at[2 * s]], ra, sem)
            pltpu.async_copy(table_hbm.at[ids_v.at[2 * s + 1]], rb, sem)

        def drain(s, ra, rb, sem):
            pltpu.make_async_copy(table_hbm.at[ids_v.at[2 * s]], ra, sem).wait()
            pltpu.make_async_copy(table_hbm.at[ids_v.at[2 * s + 1]], rb,
                                  sem).wait()

        def accum(buf, carry):
            def inner(i, c):
                a0, a1, a2, a3 = c
                for j in range(0, 20, 2):
                    r = i * 20 + j
                    a0 = a0 + buf[r, pl.ds(0, HALF)]
                    a1 = a1 + buf[r, pl.ds(HALF, HALF)]
                    a2 = a2 + buf[r + 1, pl.ds(0, HALF)]
                    a3 = a3 + buf[r + 1, pl.ds(HALF, HALF)]
                return (a0, a1, a2, a3)

            return lax.fori_loop(0, G // 20, inner, carry)

        def do_sample(s, ra, rb, sem):
            drain(s, ra, rb, sem)
            z = jnp.zeros((HALF,), jnp.float32)
            c = accum(ra, (z, z, z, z))
            c = accum(rb, c)
            sums_v[s, pl.ds(0, HALF)] = c[0] + c[2]
            sums_v[s, pl.ds(HALF, HALF)] = c[1] + c[3]

        fire(0, r0a, r0b, sem0)

        def loop_body(t, _):
            s0 = 2 * t
            fire(s0 + 1, r1a, r1b, sem1)
            do_sample(s0, r0a, r0b, sem0)

            @pl.when(s0 + 2 < SPW)
            def _():
                fire(s0 + 2, r0a, r0b, sem0)

            do_sample(s0 + 1, r1a, r1b, sem1)
            return 0

        lax.fori_loop(0, SPW // 2, loop_body, 0)
        pltpu.sync_copy(sums_v, out_hbm.at[pl.ds(wid * SPW, SPW)])

    return body(ids2d, table)


def _tc_head(sums, tt, type_table, Wp, bp):
    def body(sums_ref, tt_ref, type_ref, w_ref, b_ref, out_ref):
        c1 = jnp.sum(tt_ref[...].astype(jnp.float32), axis=1, keepdims=True)
        t0 = type_ref[0:1, :]
        t1 = type_ref[1:2, :]
        inv_s = jnp.float32(1.0 / S)
        pooled = (sums_ref[...] + (jnp.float32(S) - c1) * t0 + c1 * t1) * inv_s
        out_ref[...] = (
            jnp.dot(pooled, w_ref[...], preferred_element_type=jnp.float32)
            + b_ref[...]
        )

    return pl.pallas_call(
        body,
        out_shape=jax.ShapeDtypeStruct((B, 128), jnp.float32),
    )(sums, tt, type_table, Wp, bp)


def kernel(input_ids, attention_mask, token_type_ids, emb_table, type_table,
           W, b):
    del attention_mask
    ids2d = input_ids.astype(jnp.int32).reshape(NW * 2 * SPW, G)
    sums = _sc_emb_sum(ids2d, emb_table)
    Wp = jnp.pad(W.astype(jnp.float32), ((0, 0), (0, 128 - L)))
    bp = jnp.pad(b.astype(jnp.float32), (0, 128 - L)).reshape(1, 128)
    logits = _tc_head(sums, token_type_ids.astype(jnp.int32), type_table, Wp,
                      bp)
    return logits[:, :L]
